# baseline (speedup 1.0000x reference)
import numpy as np
import jax
import jax.numpy as jnp
from jax.experimental import pallas as pl
from jax.experimental.pallas import tpu as pltpu

_NUM_CLASSES = 1000
_B, _C, _W, _H = 64, 3, 224, 224
_ST = _B // 2
_HF = _ST // 2


def _constants():
    rng = np.random.default_rng(0)
    lam1 = float(rng.beta(0.2, 0.2))
    idx1 = rng.permutation(_HF)
    lam2 = float(rng.beta(1.0, 1.0))
    cx2 = int(rng.integers(_W))
    cy2 = int(rng.integers(_H))
    idx2 = rng.permutation(_ST - _HF)
    lam3 = float(rng.beta(0.2, 0.2))
    idx3 = rng.permutation(_HF)
    lam4 = float(rng.beta(1.0, 1.0))
    cx4 = int(rng.integers(_W))
    cy4 = int(rng.integers(_H))
    idx4 = rng.permutation((_B - _ST) - _HF)

    def cut_box(lam0, cx, cy):
        cut_rat = np.sqrt(1.0 - lam0)
        cut_w = int(_W * cut_rat)
        cut_h = int(_H * cut_rat)
        bbx1 = int(np.clip(cx - cut_w // 2, 0, _W))
        bby1 = int(np.clip(cy - cut_h // 2, 0, _H))
        bbx2 = int(np.clip(cx + cut_w // 2, 0, _W))
        bby2 = int(np.clip(cy + cut_h // 2, 0, _H))
        lam = 1.0 - (bbx2 - bbx1) * (bby2 - bby1) / float(_W * _H)
        return (bbx1, bby1, bbx2, bby2), lam

    box2, lame2 = cut_box(lam2, cx2, cy2)
    box4, lame4 = cut_box(lam4, cx4, cy4)

    masks = np.empty((4, _W, _H), np.float32)
    masks[0] = lam1
    masks[2] = lam3
    for blk, (bbx1, bby1, bbx2, bby2) in ((1, box2), (3, box4)):
        m = np.ones((_W, _H), np.float32)
        m[bbx1:bbx2, bby1:bby2] = 0.0
        masks[blk] = m

    lam_eff = np.array([lam1, lame2, lam3, lame4], np.float32)
    perm = np.concatenate(
        [idx1, _HF + idx2, _ST + idx3, _ST + _HF + idx4]
    ).astype(np.int32)
    return masks, lam_eff, perm


_MASKS_NP, _LAM_EFF, _PERM_NP = _constants()


def _body(a_ref, b_ref, lab_ref, xa_ref, xb_ref, m_ref, oi_ref, ol_ref):
    i = pl.program_id(0)
    m = m_ref[0]
    oi_ref[0, 0] = m * xa_ref[0, 0] + (1.0 - m) * xb_ref[0, 0]

    la = lab_ref[a_ref[i]]
    lb = lab_ref[b_ref[i]]
    blk = i // _HF
    lam = jnp.where(
        blk == 0,
        _LAM_EFF[0],
        jnp.where(blk == 1, _LAM_EFF[1],
                  jnp.where(blk == 2, _LAM_EFF[2], _LAM_EFF[3])),
    ).astype(jnp.float32)
    iota = jax.lax.broadcasted_iota(jnp.int32, (1, 1, _NUM_CLASSES), 2)
    ol_ref[...] = (lam * (iota == la).astype(jnp.float32)
                   + (1.0 - lam) * (iota == lb).astype(jnp.float32))


def kernel(images, labels):
    key = (labels == (_NUM_CLASSES - 1)).astype(jnp.int32)
    order = jnp.argsort(key, stable=True).astype(jnp.int32)
    a_idx = order
    b_idx = order[jnp.asarray(_PERM_NP)]
    labels32 = labels.astype(jnp.int32)
    masks = jnp.asarray(_MASKS_NP)

    grid_spec = pltpu.PrefetchScalarGridSpec(
        num_scalar_prefetch=3,
        grid=(_B, _C),
        in_specs=[
            pl.BlockSpec((1, 1, _W, _H), lambda i, c, a, b, l: (a[i], c, 0, 0)),
            pl.BlockSpec((1, 1, _W, _H), lambda i, c, a, b, l: (b[i], c, 0, 0)),
            pl.BlockSpec((1, _W, _H), lambda i, c, a, b, l: (i // _HF, 0, 0)),
        ],
        out_specs=[
            pl.BlockSpec((1, 1, _W, _H), lambda i, c, a, b, l: (i, c, 0, 0)),
            pl.BlockSpec((1, 1, _NUM_CLASSES), lambda i, c, a, b, l: (i, 0, 0)),
        ],
    )
    out_img, out_lab = pl.pallas_call(
        _body,
        grid_spec=grid_spec,
        out_shape=[
            jax.ShapeDtypeStruct((_B, _C, _W, _H), jnp.float32),
            jax.ShapeDtypeStruct((_B, 1, _NUM_CLASSES), jnp.float32),
        ],
    )(a_idx, b_idx, labels32, images, images, masks)
    return (out_img, out_lab.reshape(_B, _NUM_CLASSES))


# speedup vs baseline: 2.4041x; 2.4041x over previous
import numpy as np
import jax
import jax.numpy as jnp
from jax.experimental import pallas as pl
from jax.experimental.pallas import tpu as pltpu

_NUM_CLASSES = 1000
_B, _C, _W, _H = 64, 3, 224, 224
_ST = _B // 2
_HF = _ST // 2


def _constants():
    rng = np.random.default_rng(0)
    lam1 = float(rng.beta(0.2, 0.2))
    idx1 = rng.permutation(_HF)
    lam2 = float(rng.beta(1.0, 1.0))
    cx2 = int(rng.integers(_W))
    cy2 = int(rng.integers(_H))
    idx2 = rng.permutation(_ST - _HF)
    lam3 = float(rng.beta(0.2, 0.2))
    idx3 = rng.permutation(_HF)
    lam4 = float(rng.beta(1.0, 1.0))
    cx4 = int(rng.integers(_W))
    cy4 = int(rng.integers(_H))
    idx4 = rng.permutation((_B - _ST) - _HF)

    def cut_box(lam0, cx, cy):
        cut_rat = np.sqrt(1.0 - lam0)
        cut_w = int(_W * cut_rat)
        cut_h = int(_H * cut_rat)
        bbx1 = int(np.clip(cx - cut_w // 2, 0, _W))
        bby1 = int(np.clip(cy - cut_h // 2, 0, _H))
        bbx2 = int(np.clip(cx + cut_w // 2, 0, _W))
        bby2 = int(np.clip(cy + cut_h // 2, 0, _H))
        lam = 1.0 - (bbx2 - bbx1) * (bby2 - bby1) / float(_W * _H)
        return (bbx1, bby1, bbx2, bby2), lam

    box2, lame2 = cut_box(lam2, cx2, cy2)
    box4, lame4 = cut_box(lam4, cx4, cy4)

    masks = np.empty((4, _W, _H), np.float32)
    masks[0] = lam1
    masks[2] = lam3
    for blk, (bbx1, bby1, bbx2, bby2) in ((1, box2), (3, box4)):
        m = np.ones((_W, _H), np.float32)
        m[bbx1:bbx2, bby1:bby2] = 0.0
        masks[blk] = m

    lam_eff = np.array([lam1, lame2, lam3, lame4], np.float32)
    perm = np.concatenate(
        [idx1, _HF + idx2, _ST + idx3, _ST + _HF + idx4]
    ).astype(np.int32)
    return masks, lam_eff, perm


_MASKS_NP, _LAM_EFF, _PERM_NP = _constants()


_ROWS = 2


def _body(a_ref, b_ref, lab_ref, *refs):
    xa_refs = refs[:_ROWS]
    xb_refs = refs[_ROWS:2 * _ROWS]
    m_ref = refs[2 * _ROWS]
    oi_ref, ol_ref = refs[2 * _ROWS + 1], refs[2 * _ROWS + 2]

    i = pl.program_id(0)
    m = m_ref[0]
    iota = jax.lax.broadcasted_iota(jnp.int32, (1, _NUM_CLASSES), 1)
    blk = (i * _ROWS) // _HF
    lam = jnp.where(
        blk == 0,
        _LAM_EFF[0],
        jnp.where(blk == 1, _LAM_EFF[1],
                  jnp.where(blk == 2, _LAM_EFF[2], _LAM_EFF[3])),
    ).astype(jnp.float32)
    for j in range(_ROWS):
        oi_ref[j] = m[None] * xa_refs[j][0] + (1.0 - m)[None] * xb_refs[j][0]
        la = lab_ref[a_ref[i * _ROWS + j]]
        lb = lab_ref[b_ref[i * _ROWS + j]]
        ol_ref[j] = (lam * (iota == la).astype(jnp.float32)
                     + (1.0 - lam) * (iota == lb).astype(jnp.float32))


def kernel(images, labels):
    key = (labels == (_NUM_CLASSES - 1)).astype(jnp.int32)
    order = jnp.argsort(key, stable=True).astype(jnp.int32)
    a_idx = order
    b_idx = order[jnp.asarray(_PERM_NP)]
    labels32 = labels.astype(jnp.int32)
    masks = jnp.asarray(_MASKS_NP)

    def a_map(j):
        return lambda i, a, b, l: (a[i * _ROWS + j], 0, 0, 0)

    def b_map(j):
        return lambda i, a, b, l: (b[i * _ROWS + j], 0, 0, 0)

    img_spec = [pl.BlockSpec((1, _C, _W, _H), a_map(j)) for j in range(_ROWS)]
    img_spec += [pl.BlockSpec((1, _C, _W, _H), b_map(j)) for j in range(_ROWS)]

    grid_spec = pltpu.PrefetchScalarGridSpec(
        num_scalar_prefetch=3,
        grid=(_B // _ROWS,),
        in_specs=img_spec + [
            pl.BlockSpec((1, _W, _H),
                         lambda i, a, b, l: ((i * _ROWS) // _HF, 0, 0)),
        ],
        out_specs=[
            pl.BlockSpec((_ROWS, _C, _W, _H), lambda i, a, b, l: (i, 0, 0, 0)),
            pl.BlockSpec((_ROWS, 1, _NUM_CLASSES),
                         lambda i, a, b, l: (i, 0, 0)),
        ],
    )
    out_img, out_lab = pl.pallas_call(
        _body,
        grid_spec=grid_spec,
        out_shape=[
            jax.ShapeDtypeStruct((_B, _C, _W, _H), jnp.float32),
            jax.ShapeDtypeStruct((_B, 1, _NUM_CLASSES), jnp.float32),
        ],
    )(a_idx, b_idx, labels32, *([images] * (2 * _ROWS)), masks)
    return (out_img, out_lab.reshape(_B, _NUM_CLASSES))


# speedup vs baseline: 2.6226x; 1.0909x over previous
import numpy as np
import jax
import jax.numpy as jnp
from jax.experimental import pallas as pl
from jax.experimental.pallas import tpu as pltpu

_NUM_CLASSES = 1000
_B, _C, _W, _H = 64, 3, 224, 224
_ST = _B // 2
_HF = _ST // 2


def _constants():
    rng = np.random.default_rng(0)
    lam1 = float(rng.beta(0.2, 0.2))
    idx1 = rng.permutation(_HF)
    lam2 = float(rng.beta(1.0, 1.0))
    cx2 = int(rng.integers(_W))
    cy2 = int(rng.integers(_H))
    idx2 = rng.permutation(_ST - _HF)
    lam3 = float(rng.beta(0.2, 0.2))
    idx3 = rng.permutation(_HF)
    lam4 = float(rng.beta(1.0, 1.0))
    cx4 = int(rng.integers(_W))
    cy4 = int(rng.integers(_H))
    idx4 = rng.permutation((_B - _ST) - _HF)

    def cut_box(lam0, cx, cy):
        cut_rat = np.sqrt(1.0 - lam0)
        cut_w = int(_W * cut_rat)
        cut_h = int(_H * cut_rat)
        bbx1 = int(np.clip(cx - cut_w // 2, 0, _W))
        bby1 = int(np.clip(cy - cut_h // 2, 0, _H))
        bbx2 = int(np.clip(cx + cut_w // 2, 0, _W))
        bby2 = int(np.clip(cy + cut_h // 2, 0, _H))
        lam = 1.0 - (bbx2 - bbx1) * (bby2 - bby1) / float(_W * _H)
        return (bbx1, bby1, bbx2, bby2), lam

    box2, lame2 = cut_box(lam2, cx2, cy2)
    box4, lame4 = cut_box(lam4, cx4, cy4)

    masks = np.empty((4, _W, _H), np.float32)
    masks[0] = lam1
    masks[2] = lam3
    for blk, (bbx1, bby1, bbx2, bby2) in ((1, box2), (3, box4)):
        m = np.ones((_W, _H), np.float32)
        m[bbx1:bbx2, bby1:bby2] = 0.0
        masks[blk] = m

    lam_eff = np.array([lam1, lame2, lam3, lame4], np.float32)
    perm = np.concatenate(
        [idx1, _HF + idx2, _ST + idx3, _ST + _HF + idx4]
    ).astype(np.int32)
    return masks, lam_eff, perm


_MASKS_NP, _LAM_EFF, _PERM_NP = _constants()


_ROWS = 4


def _body(a_ref, b_ref, lab_ref, *refs):
    xa_refs = refs[:_ROWS]
    xb_refs = refs[_ROWS:2 * _ROWS]
    m_ref = refs[2 * _ROWS]
    oi_ref, ol_ref = refs[2 * _ROWS + 1], refs[2 * _ROWS + 2]

    i = pl.program_id(0)
    m = m_ref[0]
    iota = jax.lax.broadcasted_iota(jnp.int32, (1, _NUM_CLASSES), 1)
    blk = (i * _ROWS) // _HF
    lam = jnp.where(
        blk == 0,
        _LAM_EFF[0],
        jnp.where(blk == 1, _LAM_EFF[1],
                  jnp.where(blk == 2, _LAM_EFF[2], _LAM_EFF[3])),
    ).astype(jnp.float32)
    for j in range(_ROWS):
        oi_ref[j] = m[None] * xa_refs[j][0] + (1.0 - m)[None] * xb_refs[j][0]
        la = lab_ref[a_ref[i * _ROWS + j]]
        lb = lab_ref[b_ref[i * _ROWS + j]]
        ol_ref[j] = (lam * (iota == la).astype(jnp.float32)
                     + (1.0 - lam) * (iota == lb).astype(jnp.float32))


def kernel(images, labels):
    key = (labels == (_NUM_CLASSES - 1)).astype(jnp.int32)
    order = jnp.argsort(key, stable=True).astype(jnp.int32)
    a_idx = order
    b_idx = order[jnp.asarray(_PERM_NP)]
    labels32 = labels.astype(jnp.int32)
    masks = jnp.asarray(_MASKS_NP)

    def a_map(j):
        return lambda i, a, b, l: (a[i * _ROWS + j], 0, 0, 0)

    def b_map(j):
        return lambda i, a, b, l: (b[i * _ROWS + j], 0, 0, 0)

    img_spec = [pl.BlockSpec((1, _C, _W, _H), a_map(j)) for j in range(_ROWS)]
    img_spec += [pl.BlockSpec((1, _C, _W, _H), b_map(j)) for j in range(_ROWS)]

    grid_spec = pltpu.PrefetchScalarGridSpec(
        num_scalar_prefetch=3,
        grid=(_B // _ROWS,),
        in_specs=img_spec + [
            pl.BlockSpec((1, _W, _H),
                         lambda i, a, b, l: ((i * _ROWS) // _HF, 0, 0)),
        ],
        out_specs=[
            pl.BlockSpec((_ROWS, _C, _W, _H), lambda i, a, b, l: (i, 0, 0, 0)),
            pl.BlockSpec((_ROWS, 1, _NUM_CLASSES),
                         lambda i, a, b, l: (i, 0, 0)),
        ],
    )
    out_img, out_lab = pl.pallas_call(
        _body,
        grid_spec=grid_spec,
        out_shape=[
            jax.ShapeDtypeStruct((_B, _C, _W, _H), jnp.float32),
            jax.ShapeDtypeStruct((_B, 1, _NUM_CLASSES), jnp.float32),
        ],
    )(a_idx, b_idx, labels32, *([images] * (2 * _ROWS)), masks)
    return (out_img, out_lab.reshape(_B, _NUM_CLASSES))


# speedup vs baseline: 2.6400x; 1.0066x over previous
import numpy as np
import jax
import jax.numpy as jnp
from jax.experimental import pallas as pl
from jax.experimental.pallas import tpu as pltpu

_NUM_CLASSES = 1000
_B, _C, _W, _H = 64, 3, 224, 224
_ST = _B // 2
_HF = _ST // 2


def _constants():
    rng = np.random.default_rng(0)
    lam1 = float(rng.beta(0.2, 0.2))
    idx1 = rng.permutation(_HF)
    lam2 = float(rng.beta(1.0, 1.0))
    cx2 = int(rng.integers(_W))
    cy2 = int(rng.integers(_H))
    idx2 = rng.permutation(_ST - _HF)
    lam3 = float(rng.beta(0.2, 0.2))
    idx3 = rng.permutation(_HF)
    lam4 = float(rng.beta(1.0, 1.0))
    cx4 = int(rng.integers(_W))
    cy4 = int(rng.integers(_H))
    idx4 = rng.permutation((_B - _ST) - _HF)

    def cut_box(lam0, cx, cy):
        cut_rat = np.sqrt(1.0 - lam0)
        cut_w = int(_W * cut_rat)
        cut_h = int(_H * cut_rat)
        bbx1 = int(np.clip(cx - cut_w // 2, 0, _W))
        bby1 = int(np.clip(cy - cut_h // 2, 0, _H))
        bbx2 = int(np.clip(cx + cut_w // 2, 0, _W))
        bby2 = int(np.clip(cy + cut_h // 2, 0, _H))
        lam = 1.0 - (bbx2 - bbx1) * (bby2 - bby1) / float(_W * _H)
        return (bbx1, bby1, bbx2, bby2), lam

    box2, lame2 = cut_box(lam2, cx2, cy2)
    box4, lame4 = cut_box(lam4, cx4, cy4)

    masks = np.empty((4, _W, _H), np.float32)
    masks[0] = lam1
    masks[2] = lam3
    for blk, (bbx1, bby1, bbx2, bby2) in ((1, box2), (3, box4)):
        m = np.ones((_W, _H), np.float32)
        m[bbx1:bbx2, bby1:bby2] = 0.0
        masks[blk] = m

    lam_eff = np.array([lam1, lame2, lam3, lame4], np.float32)
    perm = np.concatenate(
        [idx1, _HF + idx2, _ST + idx3, _ST + _HF + idx4]
    ).astype(np.int32)
    return masks, lam_eff, perm


_MASKS_NP, _LAM_EFF, _PERM_NP = _constants()


_ROWS = 8


def _body(a_ref, b_ref, lab_ref, *refs):
    xa_refs = refs[:_ROWS]
    xb_refs = refs[_ROWS:2 * _ROWS]
    m_ref = refs[2 * _ROWS]
    oi_ref, ol_ref = refs[2 * _ROWS + 1], refs[2 * _ROWS + 2]

    i = pl.program_id(0)
    m = m_ref[0]
    iota = jax.lax.broadcasted_iota(jnp.int32, (1, _NUM_CLASSES), 1)
    blk = (i * _ROWS) // _HF
    lam = jnp.where(
        blk == 0,
        _LAM_EFF[0],
        jnp.where(blk == 1, _LAM_EFF[1],
                  jnp.where(blk == 2, _LAM_EFF[2], _LAM_EFF[3])),
    ).astype(jnp.float32)
    for j in range(_ROWS):
        oi_ref[j] = m[None] * xa_refs[j][0] + (1.0 - m)[None] * xb_refs[j][0]
        la = lab_ref[a_ref[i * _ROWS + j]]
        lb = lab_ref[b_ref[i * _ROWS + j]]
        ol_ref[j] = (lam * (iota == la).astype(jnp.float32)
                     + (1.0 - lam) * (iota == lb).astype(jnp.float32))


def kernel(images, labels):
    key = (labels == (_NUM_CLASSES - 1)).astype(jnp.int32)
    order = jnp.argsort(key, stable=True).astype(jnp.int32)
    a_idx = order
    b_idx = order[jnp.asarray(_PERM_NP)]
    labels32 = labels.astype(jnp.int32)
    masks = jnp.asarray(_MASKS_NP)

    def a_map(j):
        return lambda i, a, b, l: (a[i * _ROWS + j], 0, 0, 0)

    def b_map(j):
        return lambda i, a, b, l: (b[i * _ROWS + j], 0, 0, 0)

    img_spec = [pl.BlockSpec((1, _C, _W, _H), a_map(j)) for j in range(_ROWS)]
    img_spec += [pl.BlockSpec((1, _C, _W, _H), b_map(j)) for j in range(_ROWS)]

    grid_spec = pltpu.PrefetchScalarGridSpec(
        num_scalar_prefetch=3,
        grid=(_B // _ROWS,),
        in_specs=img_spec + [
            pl.BlockSpec((1, _W, _H),
                         lambda i, a, b, l: ((i * _ROWS) // _HF, 0, 0)),
        ],
        out_specs=[
            pl.BlockSpec((_ROWS, _C, _W, _H), lambda i, a, b, l: (i, 0, 0, 0)),
            pl.BlockSpec((_ROWS, 1, _NUM_CLASSES),
                         lambda i, a, b, l: (i, 0, 0)),
        ],
    )
    out_img, out_lab = pl.pallas_call(
        _body,
        grid_spec=grid_spec,
        out_shape=[
            jax.ShapeDtypeStruct((_B, _C, _W, _H), jnp.float32),
            jax.ShapeDtypeStruct((_B, 1, _NUM_CLASSES), jnp.float32),
        ],
    )(a_idx, b_idx, labels32, *([images] * (2 * _ROWS)), masks)
    return (out_img, out_lab.reshape(_B, _NUM_CLASSES))


# speedup vs baseline: 2.6677x; 1.0105x over previous
import numpy as np
import jax
import jax.numpy as jnp
from jax.experimental import pallas as pl
from jax.experimental.pallas import tpu as pltpu

_NUM_CLASSES = 1000
_B, _C, _W, _H = 64, 3, 224, 224
_ST = _B // 2
_HF = _ST // 2


def _constants():
    rng = np.random.default_rng(0)
    lam1 = float(rng.beta(0.2, 0.2))
    idx1 = rng.permutation(_HF)
    lam2 = float(rng.beta(1.0, 1.0))
    cx2 = int(rng.integers(_W))
    cy2 = int(rng.integers(_H))
    idx2 = rng.permutation(_ST - _HF)
    lam3 = float(rng.beta(0.2, 0.2))
    idx3 = rng.permutation(_HF)
    lam4 = float(rng.beta(1.0, 1.0))
    cx4 = int(rng.integers(_W))
    cy4 = int(rng.integers(_H))
    idx4 = rng.permutation((_B - _ST) - _HF)

    def cut_box(lam0, cx, cy):
        cut_rat = np.sqrt(1.0 - lam0)
        cut_w = int(_W * cut_rat)
        cut_h = int(_H * cut_rat)
        bbx1 = int(np.clip(cx - cut_w // 2, 0, _W))
        bby1 = int(np.clip(cy - cut_h // 2, 0, _H))
        bbx2 = int(np.clip(cx + cut_w // 2, 0, _W))
        bby2 = int(np.clip(cy + cut_h // 2, 0, _H))
        lam = 1.0 - (bbx2 - bbx1) * (bby2 - bby1) / float(_W * _H)
        return (bbx1, bby1, bbx2, bby2), lam

    box2, lame2 = cut_box(lam2, cx2, cy2)
    box4, lame4 = cut_box(lam4, cx4, cy4)

    lam_eff = np.array([lam1, lame2, lam3, lame4], np.float32)
    perm = np.concatenate(
        [idx1, _HF + idx2, _ST + idx3, _ST + _HF + idx4]
    ).astype(np.int32)
    return box2, box4, lam_eff, perm


_BOX2, _BOX4, _LAM_EFF, _PERM_NP = _constants()


_ROWS = 8

_BOXES = {1: _BOX2, 3: _BOX4}


def _body(a_ref, b_ref, lab_ref, *refs):
    xa_refs = refs[:_ROWS]
    xb_refs = refs[_ROWS:2 * _ROWS]
    oi_ref, ol_ref = refs[2 * _ROWS], refs[2 * _ROWS + 1]

    i = pl.program_id(0)
    iota = jax.lax.broadcasted_iota(jnp.int32, (1, _NUM_CLASSES), 1)
    steps_per_blk = _HF // _ROWS

    for blk in range(4):
        lam = float(_LAM_EFF[blk])

        @pl.when(i // steps_per_blk == blk)
        def _(blk=blk, lam=lam):
            if blk in _BOXES:
                bbx1, bby1, bbx2, bby2 = _BOXES[blk]
                xi = jax.lax.broadcasted_iota(jnp.int32, (_W, _H), 0)
                yi = jax.lax.broadcasted_iota(jnp.int32, (_W, _H), 1)
                inside = ((xi >= bbx1) & (xi < bbx2)
                          & (yi >= bby1) & (yi < bby2))
                for j in range(_ROWS):
                    oi_ref[j] = jnp.where(inside[None], xb_refs[j][0],
                                          xa_refs[j][0])
            else:
                for j in range(_ROWS):
                    oi_ref[j] = (lam * xa_refs[j][0]
                                 + (1.0 - lam) * xb_refs[j][0])
            for j in range(_ROWS):
                la = lab_ref[a_ref[i * _ROWS + j]]
                lb = lab_ref[b_ref[i * _ROWS + j]]
                ol_ref[j] = (lam * (iota == la).astype(jnp.float32)
                             + (1.0 - lam) * (iota == lb).astype(jnp.float32))


def kernel(images, labels):
    key = (labels == (_NUM_CLASSES - 1)).astype(jnp.int32)
    order = jnp.argsort(key, stable=True).astype(jnp.int32)
    a_idx = order
    b_idx = order[jnp.asarray(_PERM_NP)]
    labels32 = labels.astype(jnp.int32)

    def a_map(j):
        return lambda i, a, b, l: (a[i * _ROWS + j], 0, 0, 0)

    def b_map(j):
        return lambda i, a, b, l: (b[i * _ROWS + j], 0, 0, 0)

    img_spec = [pl.BlockSpec((1, _C, _W, _H), a_map(j)) for j in range(_ROWS)]
    img_spec += [pl.BlockSpec((1, _C, _W, _H), b_map(j)) for j in range(_ROWS)]

    grid_spec = pltpu.PrefetchScalarGridSpec(
        num_scalar_prefetch=3,
        grid=(_B // _ROWS,),
        in_specs=img_spec,
        out_specs=[
            pl.BlockSpec((_ROWS, _C, _W, _H), lambda i, a, b, l: (i, 0, 0, 0)),
            pl.BlockSpec((_ROWS, 1, _NUM_CLASSES),
                         lambda i, a, b, l: (i, 0, 0)),
        ],
    )
    out_img, out_lab = pl.pallas_call(
        _body,
        grid_spec=grid_spec,
        out_shape=[
            jax.ShapeDtypeStruct((_B, _C, _W, _H), jnp.float32),
            jax.ShapeDtypeStruct((_B, 1, _NUM_CLASSES), jnp.float32),
        ],
    )(a_idx, b_idx, labels32, *([images] * (2 * _ROWS)))
    return (out_img, out_lab.reshape(_B, _NUM_CLASSES))


# speedup vs baseline: 3.6095x; 1.3530x over previous
import numpy as np
import jax
import jax.numpy as jnp
from jax.experimental import pallas as pl
from jax.experimental.pallas import tpu as pltpu

_NUM_CLASSES = 1000
_B, _C, _W, _H = 64, 3, 224, 224
_ST = _B // 2
_HF = _ST // 2


def _constants():
    rng = np.random.default_rng(0)
    lam1 = float(rng.beta(0.2, 0.2))
    idx1 = rng.permutation(_HF)
    lam2 = float(rng.beta(1.0, 1.0))
    cx2 = int(rng.integers(_W))
    cy2 = int(rng.integers(_H))
    idx2 = rng.permutation(_ST - _HF)
    lam3 = float(rng.beta(0.2, 0.2))
    idx3 = rng.permutation(_HF)
    lam4 = float(rng.beta(1.0, 1.0))
    cx4 = int(rng.integers(_W))
    cy4 = int(rng.integers(_H))
    idx4 = rng.permutation((_B - _ST) - _HF)

    def cut_box(lam0, cx, cy):
        cut_rat = np.sqrt(1.0 - lam0)
        cut_w = int(_W * cut_rat)
        cut_h = int(_H * cut_rat)
        bbx1 = int(np.clip(cx - cut_w // 2, 0, _W))
        bby1 = int(np.clip(cy - cut_h // 2, 0, _H))
        bbx2 = int(np.clip(cx + cut_w // 2, 0, _W))
        bby2 = int(np.clip(cy + cut_h // 2, 0, _H))
        lam = 1.0 - (bbx2 - bbx1) * (bby2 - bby1) / float(_W * _H)
        return (bbx1, bby1, bbx2, bby2), lam

    box2, lame2 = cut_box(lam2, cx2, cy2)
    box4, lame4 = cut_box(lam4, cx4, cy4)

    lam_eff = np.array([lam1, lame2, lam3, lame4], np.float32)
    perm = np.concatenate(
        [idx1, _HF + idx2, _ST + idx3, _ST + _HF + idx4]
    ).astype(np.int32)
    return box2, box4, lam_eff, perm


_BOX2, _BOX4, _LAM_EFF, _PERM_NP = _constants()


_ROWS = 8

_BOXES = {1: _BOX2, 3: _BOX4}


def _body(a_ref, b_ref, lab_ref, img_ref, oi_ref, ol_ref):
    i = pl.program_id(0)
    iota = jax.lax.broadcasted_iota(jnp.int32, (1, _NUM_CLASSES), 1)
    steps_per_blk = _HF // _ROWS

    for blk in range(4):
        lam = float(_LAM_EFF[blk])

        @pl.when(i // steps_per_blk == blk)
        def _(blk=blk, lam=lam):
            if blk in _BOXES:
                bbx1, bby1, bbx2, bby2 = _BOXES[blk]
                xi = jax.lax.broadcasted_iota(jnp.int32, (_W, _H), 0)
                yi = jax.lax.broadcasted_iota(jnp.int32, (_W, _H), 1)
                inside = ((xi >= bbx1) & (xi < bbx2)
                          & (yi >= bby1) & (yi < bby2))
                for j in range(_ROWS):
                    xa = img_ref[a_ref[i * _ROWS + j]]
                    xb = img_ref[b_ref[i * _ROWS + j]]
                    oi_ref[j] = jnp.where(inside[None], xb, xa)
            else:
                for j in range(_ROWS):
                    xa = img_ref[a_ref[i * _ROWS + j]]
                    xb = img_ref[b_ref[i * _ROWS + j]]
                    oi_ref[j] = lam * xa + (1.0 - lam) * xb
            for j in range(_ROWS):
                la = lab_ref[a_ref[i * _ROWS + j]]
                lb = lab_ref[b_ref[i * _ROWS + j]]
                ol_ref[j] = (lam * (iota == la).astype(jnp.float32)
                             + (1.0 - lam) * (iota == lb).astype(jnp.float32))


def kernel(images, labels):
    key = (labels == (_NUM_CLASSES - 1)).astype(jnp.int32)
    order = jnp.argsort(key, stable=True).astype(jnp.int32)
    a_idx = order
    b_idx = order[jnp.asarray(_PERM_NP)]
    labels32 = labels.astype(jnp.int32)

    grid_spec = pltpu.PrefetchScalarGridSpec(
        num_scalar_prefetch=3,
        grid=(_B // _ROWS,),
        in_specs=[
            pl.BlockSpec((_B, _C, _W, _H), lambda i, a, b, l: (0, 0, 0, 0)),
        ],
        out_specs=[
            pl.BlockSpec((_ROWS, _C, _W, _H), lambda i, a, b, l: (i, 0, 0, 0)),
            pl.BlockSpec((_ROWS, 1, _NUM_CLASSES),
                         lambda i, a, b, l: (i, 0, 0)),
        ],
    )
    out_img, out_lab = pl.pallas_call(
        _body,
        grid_spec=grid_spec,
        out_shape=[
            jax.ShapeDtypeStruct((_B, _C, _W, _H), jnp.float32),
            jax.ShapeDtypeStruct((_B, 1, _NUM_CLASSES), jnp.float32),
        ],
        compiler_params=pltpu.CompilerParams(
            vmem_limit_bytes=100 * 1024 * 1024,
        ),
    )(a_idx, b_idx, labels32, images)
    return (out_img, out_lab.reshape(_B, _NUM_CLASSES))
